# baseline (device time: 46408 ns/iter reference)
import jax
import jax.numpy as jnp
from jax import lax
from jax.experimental import pallas as pl
from jax.experimental.pallas import tpu as pltpu

M, N = 2048, 1024
HALF = M // 2
QTR = M // 4
EIG = M // 8

BF16 = jnp.bfloat16


def kernel(x):
    def body(x_hbm, out_hbm, xf_ref, acc_ref, sx_ref, sy_ref, r1x_ref,
             r1y_ref, r2y_ref, r2x_ref, load_sems, store_sems, send_sems,
             recv_sems):
        mx = lax.axis_index("x")
        my = lax.axis_index("y")
        xn = (1 - mx, my)
        yn = (mx, 1 - my)

        bar = pltpu.get_barrier_semaphore()
        pl.semaphore_signal(bar, inc=1, device_id=xn,
                            device_id_type=pl.DeviceIdType.MESH)
        pl.semaphore_signal(bar, inc=1, device_id=yn,
                            device_id_type=pl.DeviceIdType.MESH)
        pl.semaphore_wait(bar, 2)

        base0 = mx * QTR
        base1 = HALF + my * QTR
        e0 = base0 + my * EIG
        e1 = base1 + mx * EIG
        f0 = base0 + (1 - my) * EIG
        f1 = base1 + (1 - mx) * EIG
        offA0, offB0 = (1 - my) * EIG, my * EIG
        offA1, offB1 = (1 - mx) * EIG, mx * EIG

        load_rows = [
            (1 - mx) * QTR + offA0,
            HALF + (1 - my) * QTR + offA1,
            (1 - mx) * QTR + offB0,
            HALF + (1 - my) * QTR + offB1,
            f0,
            f1,
            e0,
            e1,
        ]
        loads = []
        for i, row in enumerate(load_rows):
            cp = pltpu.make_async_copy(
                x_hbm.at[0, 0, pl.ds(row, EIG), :],
                xf_ref.at[pl.ds(row, EIG), :],
                load_sems.at[i],
            )
            cp.start()
            loads.append(cp)

        def xb(row, nrows):
            return xf_ref[pl.ds(row, nrows), :].astype(BF16)

        def exchange(idx, src, dst, dev):
            return pltpu.make_async_remote_copy(
                src_ref=src, dst_ref=dst,
                send_sem=send_sems.at[idx], recv_sem=recv_sems.at[idx],
                device_id=dev, device_id_type=pl.DeviceIdType.MESH,
            )

        stores = []

        def store(row, nrows):
            cp = pltpu.make_async_copy(
                acc_ref.at[pl.ds(row, nrows), :],
                out_hbm.at[pl.ds(row, nrows), :],
                store_sems.at[len(stores)],
            )
            cp.start()
            stores.append(cp)

        loads[0].wait()
        sx_ref[pl.ds(offA0, EIG), :] = xb((1 - mx) * QTR + offA0, EIG)
        rd0 = exchange(0, sx_ref.at[pl.ds(offA0, EIG), :],
                       r1x_ref.at[pl.ds(offA0, EIG), :], xn)
        rd0.start()
        loads[1].wait()
        sy_ref[pl.ds(offA1, EIG), :] = xb(HALF + (1 - my) * QTR + offA1, EIG)
        rd1 = exchange(1, sy_ref.at[pl.ds(offA1, EIG), :],
                       r1y_ref.at[pl.ds(offA1, EIG), :], yn)
        rd1.start()
        loads[2].wait()
        sx_ref[pl.ds(offB0, EIG), :] = xb((1 - mx) * QTR + offB0, EIG)
        rd2 = exchange(2, sx_ref.at[pl.ds(offB0, EIG), :],
                       r1x_ref.at[pl.ds(offB0, EIG), :], xn)
        rd2.start()
        loads[3].wait()
        sy_ref[pl.ds(offB1, EIG), :] = xb(HALF + (1 - my) * QTR + offB1, EIG)
        rd3 = exchange(3, sy_ref.at[pl.ds(offB1, EIG), :],
                       r1y_ref.at[pl.ds(offB1, EIG), :], yn)
        rd3.start()

        rd0.wait_recv()
        loads[4].wait()
        acc_ref[pl.ds(f0, EIG), :] = xb(f0, EIG) + r1x_ref[pl.ds(offA0, EIG), :]
        rd4 = exchange(4, acc_ref.at[pl.ds(f0, EIG), :], r2y_ref, yn)
        rd4.start()
        rd1.wait_recv()
        loads[5].wait()
        acc_ref[pl.ds(f1, EIG), :] = xb(f1, EIG) + r1y_ref[pl.ds(offA1, EIG), :]
        rd5 = exchange(5, acc_ref.at[pl.ds(f1, EIG), :], r2x_ref, xn)
        rd5.start()
        rd2.wait_recv()
        loads[6].wait()
        acc_ref[pl.ds(e0, EIG), :] = xb(e0, EIG) + r1x_ref[pl.ds(offB0, EIG), :]
        rd3.wait_recv()
        loads[7].wait()
        acc_ref[pl.ds(e1, EIG), :] = xb(e1, EIG) + r1y_ref[pl.ds(offB1, EIG), :]

        rd4.wait_recv()
        acc_ref[pl.ds(e0, EIG), :] = acc_ref[pl.ds(e0, EIG), :] + r2y_ref[...]
        rd6 = exchange(6, acc_ref.at[pl.ds(e0, EIG), :],
                       acc_ref.at[pl.ds(e0, EIG), :], yn)
        rd6.start()
        rd5.wait_recv()
        acc_ref[pl.ds(e1, EIG), :] = acc_ref[pl.ds(e1, EIG), :] + r2x_ref[...]
        rd7 = exchange(7, acc_ref.at[pl.ds(e1, EIG), :],
                       acc_ref.at[pl.ds(e1, EIG), :], xn)
        rd7.start()
        rd8 = exchange(8, acc_ref.at[pl.ds(e0, EIG), :],
                       acc_ref.at[pl.ds(e0, EIG), :], xn)
        rd8.start()
        rd9 = exchange(9, acc_ref.at[pl.ds(e1, EIG), :],
                       acc_ref.at[pl.ds(e1, EIG), :], yn)
        rd9.start()
        store(e0, EIG)
        store(e1, EIG)

        rd6.wait_recv()
        rd10 = exchange(10, acc_ref.at[pl.ds(f0, EIG), :],
                        acc_ref.at[pl.ds(f0, EIG), :], xn)
        rd10.start()
        store(f0, EIG)
        rd7.wait_recv()
        rd11 = exchange(11, acc_ref.at[pl.ds(f1, EIG), :],
                        acc_ref.at[pl.ds(f1, EIG), :], yn)
        rd11.start()
        store(f1, EIG)

        rd8.wait_recv()
        store((1 - mx) * QTR + my * EIG, EIG)
        rd9.wait_recv()
        store(HALF + (1 - my) * QTR + mx * EIG, EIG)
        rd10.wait_recv()
        store((1 - mx) * QTR + (1 - my) * EIG, EIG)
        rd11.wait_recv()
        store(HALF + (1 - my) * QTR + (1 - mx) * EIG, EIG)

        for rd in (rd0, rd1, rd2, rd3, rd4, rd5, rd6, rd7, rd8, rd9, rd10,
                   rd11):
            rd.wait_send()
        for cp in stores:
            cp.wait()

    return pl.pallas_call(
        body,
        out_shape=jax.ShapeDtypeStruct((M, N), BF16),
        in_specs=[pl.BlockSpec(memory_space=pl.ANY)],
        out_specs=pl.BlockSpec(memory_space=pl.ANY),
        scratch_shapes=[
            pltpu.VMEM((M, N), jnp.float32),
            pltpu.VMEM((M, N), BF16),
            pltpu.VMEM((QTR, N), BF16),
            pltpu.VMEM((QTR, N), BF16),
            pltpu.VMEM((QTR, N), BF16),
            pltpu.VMEM((QTR, N), BF16),
            pltpu.VMEM((EIG, N), BF16),
            pltpu.VMEM((EIG, N), BF16),
            pltpu.SemaphoreType.DMA((8,)),
            pltpu.SemaphoreType.DMA((8,)),
            pltpu.SemaphoreType.DMA((12,)),
            pltpu.SemaphoreType.DMA((12,)),
        ],
        compiler_params=pltpu.CompilerParams(collective_id=0),
    )(x)


# device time: 43173 ns/iter; 1.0749x vs baseline; 1.0749x over previous
import jax
import jax.numpy as jnp
from jax import lax
from jax.experimental import pallas as pl
from jax.experimental.pallas import tpu as pltpu

M, N = 2048, 1024
HALF = M // 2
QTR = M // 4
EIG = M // 8
SUB = M // 16

BF16 = jnp.bfloat16


def kernel(x):
    def body(x_ref, out_ref, sx_ref, sy_ref, r1x_ref, r1y_ref, r2y_ref,
             r2x_ref, send_sems, recv_sems):
        mx = lax.axis_index("x")
        my = lax.axis_index("y")
        xn = (1 - mx, my)
        yn = (mx, 1 - my)

        bar = pltpu.get_barrier_semaphore()
        pl.semaphore_signal(bar, inc=1, device_id=xn,
                            device_id_type=pl.DeviceIdType.MESH)
        pl.semaphore_signal(bar, inc=1, device_id=yn,
                            device_id_type=pl.DeviceIdType.MESH)
        pl.semaphore_wait(bar, 2)

        def xb(row, nrows):
            return x_ref[0, 0, pl.ds(row, nrows), :].astype(BF16)

        sems = iter(range(18))

        def exchange(src, dst, dev):
            idx = next(sems)
            return pltpu.make_async_remote_copy(
                src_ref=src, dst_ref=dst,
                send_sem=send_sems.at[idx], recv_sem=recv_sems.at[idx],
                device_id=dev, device_id_type=pl.DeviceIdType.MESH,
            )

        base0 = mx * QTR
        base1 = HALF + my * QTR
        e0 = base0 + my * EIG
        e1 = base1 + mx * EIG
        f0 = base0 + (1 - my) * EIG
        f1 = base1 + (1 - mx) * EIG
        offA0, offB0 = (1 - my) * EIG, my * EIG
        offA1, offB1 = (1 - mx) * EIG, mx * EIG

        sx_ref[pl.ds(offA0, EIG), :] = xb((1 - mx) * QTR + offA0, EIG)
        rd0 = exchange(sx_ref.at[pl.ds(offA0, EIG), :],
                       r1x_ref.at[pl.ds(offA0, EIG), :], xn)
        rd0.start()
        sy_ref[pl.ds(offA1, EIG), :] = xb(HALF + (1 - my) * QTR + offA1, EIG)
        rd1 = exchange(sy_ref.at[pl.ds(offA1, EIG), :],
                       r1y_ref.at[pl.ds(offA1, EIG), :], yn)
        rd1.start()
        sx_ref[pl.ds(offB0, EIG), :] = xb((1 - mx) * QTR + offB0, EIG)
        rd2 = exchange(sx_ref.at[pl.ds(offB0, EIG), :],
                       r1x_ref.at[pl.ds(offB0, EIG), :], xn)
        rd2.start()
        sy_ref[pl.ds(offB1, EIG), :] = xb(HALF + (1 - my) * QTR + offB1, EIG)
        rd3 = exchange(sy_ref.at[pl.ds(offB1, EIG), :],
                       r1y_ref.at[pl.ds(offB1, EIG), :], yn)
        rd3.start()

        rd0.wait_recv()
        out_ref[pl.ds(f0, EIG), :] = xb(f0, EIG) + r1x_ref[pl.ds(offA0, EIG), :]
        rd4a = exchange(out_ref.at[pl.ds(f0, SUB), :],
                        r2y_ref.at[pl.ds(0, SUB), :], yn)
        rd4a.start()
        rd4b = exchange(out_ref.at[pl.ds(f0 + SUB, SUB), :],
                        r2y_ref.at[pl.ds(SUB, SUB), :], yn)
        rd4b.start()
        rd1.wait_recv()
        out_ref[pl.ds(f1, EIG), :] = xb(f1, EIG) + r1y_ref[pl.ds(offA1, EIG), :]
        rd5a = exchange(out_ref.at[pl.ds(f1, SUB), :],
                        r2x_ref.at[pl.ds(0, SUB), :], xn)
        rd5a.start()
        rd5b = exchange(out_ref.at[pl.ds(f1 + SUB, SUB), :],
                        r2x_ref.at[pl.ds(SUB, SUB), :], xn)
        rd5b.start()
        rd2.wait_recv()
        out_ref[pl.ds(e0, EIG), :] = xb(e0, EIG) + r1x_ref[pl.ds(offB0, EIG), :]
        rd3.wait_recv()
        out_ref[pl.ds(e1, EIG), :] = xb(e1, EIG) + r1y_ref[pl.ds(offB1, EIG), :]

        rd4a.wait_recv()
        out_ref[pl.ds(e0, SUB), :] = (
            out_ref[pl.ds(e0, SUB), :] + r2y_ref[pl.ds(0, SUB), :])
        rd6a = exchange(out_ref.at[pl.ds(e0, SUB), :],
                        out_ref.at[pl.ds(e0, SUB), :], yn)
        rd6a.start()
        rd5a.wait_recv()
        out_ref[pl.ds(e1, SUB), :] = (
            out_ref[pl.ds(e1, SUB), :] + r2x_ref[pl.ds(0, SUB), :])
        rd7a = exchange(out_ref.at[pl.ds(e1, SUB), :],
                        out_ref.at[pl.ds(e1, SUB), :], xn)
        rd7a.start()
        rd4b.wait_recv()
        out_ref[pl.ds(e0 + SUB, SUB), :] = (
            out_ref[pl.ds(e0 + SUB, SUB), :] + r2y_ref[pl.ds(SUB, SUB), :])
        rd6b = exchange(out_ref.at[pl.ds(e0 + SUB, SUB), :],
                        out_ref.at[pl.ds(e0 + SUB, SUB), :], yn)
        rd6b.start()
        rd5b.wait_recv()
        out_ref[pl.ds(e1 + SUB, SUB), :] = (
            out_ref[pl.ds(e1 + SUB, SUB), :] + r2x_ref[pl.ds(SUB, SUB), :])
        rd7b = exchange(out_ref.at[pl.ds(e1 + SUB, SUB), :],
                        out_ref.at[pl.ds(e1 + SUB, SUB), :], xn)
        rd7b.start()

        rd8 = exchange(out_ref.at[pl.ds(e0, EIG), :],
                       out_ref.at[pl.ds(e0, EIG), :], xn)
        rd8.start()
        rd9 = exchange(out_ref.at[pl.ds(e1, EIG), :],
                       out_ref.at[pl.ds(e1, EIG), :], yn)
        rd9.start()

        rd6a.wait_recv()
        rd10a = exchange(out_ref.at[pl.ds(f0, SUB), :],
                         out_ref.at[pl.ds(f0, SUB), :], xn)
        rd10a.start()
        rd7a.wait_recv()
        rd11a = exchange(out_ref.at[pl.ds(f1, SUB), :],
                         out_ref.at[pl.ds(f1, SUB), :], yn)
        rd11a.start()
        rd6b.wait_recv()
        rd10b = exchange(out_ref.at[pl.ds(f0 + SUB, SUB), :],
                         out_ref.at[pl.ds(f0 + SUB, SUB), :], xn)
        rd10b.start()
        rd7b.wait_recv()
        rd11b = exchange(out_ref.at[pl.ds(f1 + SUB, SUB), :],
                         out_ref.at[pl.ds(f1 + SUB, SUB), :], yn)
        rd11b.start()

        all_rds = (rd0, rd1, rd2, rd3, rd4a, rd4b, rd5a, rd5b, rd6a, rd6b,
                   rd7a, rd7b, rd8, rd9, rd10a, rd11a, rd10b, rd11b)
        for rd in (rd8, rd9, rd10a, rd11a, rd10b, rd11b):
            rd.wait_recv()
        for rd in all_rds:
            rd.wait_send()

    return pl.pallas_call(
        body,
        out_shape=jax.ShapeDtypeStruct((M, N), BF16),
        in_specs=[pl.BlockSpec(memory_space=pltpu.VMEM)],
        out_specs=pl.BlockSpec(memory_space=pltpu.VMEM),
        scratch_shapes=[
            pltpu.VMEM((QTR, N), BF16),
            pltpu.VMEM((QTR, N), BF16),
            pltpu.VMEM((QTR, N), BF16),
            pltpu.VMEM((QTR, N), BF16),
            pltpu.VMEM((EIG, N), BF16),
            pltpu.VMEM((EIG, N), BF16),
            pltpu.SemaphoreType.DMA((18,)),
            pltpu.SemaphoreType.DMA((18,)),
        ],
        compiler_params=pltpu.CompilerParams(collective_id=0),
    )(x)
